# baseline (speedup 1.0000x reference)
import functools

import jax
import jax.numpy as jnp
from jax import lax
from jax.experimental import pallas as pl
from jax.experimental.pallas import tpu as pltpu
from jax.experimental.pallas import tpu_sc as plsc

NUM_CORES = 2
NUM_SUBCORES = 16
LANES = 16
NUM_WORKERS = NUM_CORES * NUM_SUBCORES

BATCH = 16384
EMBED = 32
PACK = 128 // EMBED
B_PER_W = BATCH // NUM_WORKERS
CHUNK = 128
ID_ROWS = B_PER_W // CHUNK
HALF = B_PER_W // 2
HROWS = HALF // CHUNK


def _mf_body(uids_hbm, mids_hbm, ustage_hbm, mstage_hbm, out_hbm,
             uidv, midv, uix, mix, urows, mrows, outv, sem_u, sem_m):
  wid = lax.axis_index("s") * NUM_CORES + lax.axis_index("c")

  pltpu.sync_copy(uids_hbm.at[pl.ds(wid * ID_ROWS, ID_ROWS)], uidv)
  pltpu.sync_copy(mids_hbm.at[pl.ds(wid * ID_ROWS, ID_ROWS)], midv)

  def mkidx(r, _):
    for cc in range(CHUNK // LANES):
      col = cc * LANES
      uv = uidv[r, pl.ds(col, LANES)]
      mv = midv[r, pl.ds(col, LANES)]
      uix[r, pl.ds(col, LANES)] = ((uv >> 11) << 9) | (uv & 511)
      mix[r, pl.ds(col, LANES)] = ((mv >> 11) << 9) | (mv & 511)
    return 0

  lax.fori_loop(0, ID_ROWS, mkidx, 0)

  lane = lax.iota(jnp.int32, LANES)

  def run_pass(p, _):
    for rr in range(HROWS):
      pltpu.async_copy(
          ustage_hbm.at[uix.at[p * HROWS + rr]],
          urows.at[pl.ds(rr * CHUNK, CHUNK)], sem_u)
      pltpu.async_copy(
          mstage_hbm.at[mix.at[p * HROWS + rr]],
          mrows.at[pl.ds(rr * CHUNK, CHUNK)], sem_m)
    pltpu.make_async_copy(
        ustage_hbm.at[pl.ds(0, HALF)], urows, sem_u).wait()
    pltpu.make_async_copy(
        mstage_hbm.at[pl.ds(0, HALF)], mrows, sem_m).wait()

    def dot(g, _):
      b = p * HALF + g * LANES
      row = b // CHUNK
      col = b % CHUNK
      uvec = uidv[row, pl.ds(col, LANES)]
      mvec = midv[row, pl.ds(col, LANES)]
      acc = jnp.zeros((LANES,), jnp.float32)
      for k in range(LANES):
        e = g * LANES + k
        us = ((uvec[k] >> 9) & 3) * EMBED
        ms = ((mvec[k] >> 9) & 3) * EMBED
        u0 = urows[e, pl.ds(us, LANES)]
        u1 = urows[e, pl.ds(us + LANES, LANES)]
        m0 = mrows[e, pl.ds(ms, LANES)]
        m1 = mrows[e, pl.ds(ms + LANES, LANES)]
        s = u0 * m0 + u1 * m1
        tot = jnp.sum(s)
        acc = acc + jnp.where(lane == k, tot, jnp.float32(0))
      outv[pl.ds(b, LANES)] = acc
      return 0

    lax.fori_loop(0, HALF // LANES, dot, 0)
    return 0

  lax.fori_loop(0, 2, run_pass, 0)

  pltpu.sync_copy(outv, out_hbm.at[pl.ds(wid * B_PER_W, B_PER_W)])


RCW = 2048
QW = RCW // PACK


def _relayout_block(tab_ref, out_ref):
  x = tab_ref[...]
  eye = jnp.eye(EMBED, dtype=jnp.float32)
  parts = [
      jax.lax.dot_general(
          x[:, a * QW:(a + 1) * QW], eye, (((0,), (0,)), ((), ())),
          preferred_element_type=jnp.float32)
      for a in range(PACK)
  ]
  out_ref[...] = jnp.concatenate(parts, axis=1)


def _relayout(tab_t):
  n = tab_t.shape[1]
  grid = (n + RCW - 1) // RCW
  return pl.pallas_call(
      _relayout_block,
      grid=(grid,),
      in_specs=[pl.BlockSpec((EMBED, RCW), lambda i: (0, i))],
      out_specs=pl.BlockSpec((QW, 128), lambda i: (i, 0)),
      out_shape=jax.ShapeDtypeStruct((grid * QW, 128), jnp.float32),
  )(tab_t)


@jax.jit
def _mf(user_ids, movie_ids, user_table, movie_table):
  kern = pl.kernel(
      _mf_body,
      out_type=jax.ShapeDtypeStruct((BATCH,), jnp.float32),
      mesh=plsc.VectorSubcoreMesh(core_axis_name="c", subcore_axis_name="s"),
      scratch_types=[
          pltpu.VMEM((ID_ROWS, CHUNK), jnp.int32),
          pltpu.VMEM((ID_ROWS, CHUNK), jnp.int32),
          pltpu.VMEM((ID_ROWS, CHUNK), jnp.int32),
          pltpu.VMEM((ID_ROWS, CHUNK), jnp.int32),
          pltpu.VMEM((HALF, 128), jnp.float32),
          pltpu.VMEM((HALF, 128), jnp.float32),
          pltpu.VMEM((B_PER_W,), jnp.float32),
          pltpu.SemaphoreType.DMA,
          pltpu.SemaphoreType.DMA,
      ],
      compiler_params=pltpu.CompilerParams(needs_layout_passes=False),
  )
  uids = user_ids.astype(jnp.int32).reshape(BATCH // CHUNK, CHUNK)
  mids = movie_ids.astype(jnp.int32).reshape(BATCH // CHUNK, CHUNK)
  ustage = _relayout(jnp.swapaxes(user_table, 0, 1))
  mstage = _relayout(jnp.swapaxes(movie_table, 0, 1))
  return kern(uids, mids, ustage, mstage)


def kernel(user_ids, movie_ids, user_table, movie_table):
  return _mf(user_ids, movie_ids, user_table, movie_table)


# speedup vs baseline: 1.2739x; 1.2739x over previous
import functools

import jax
import jax.numpy as jnp
from jax import lax
from jax.experimental import pallas as pl
from jax.experimental.pallas import tpu as pltpu
from jax.experimental.pallas import tpu_sc as plsc

NUM_CORES = 2
NUM_SUBCORES = 16
LANES = 16
NUM_WORKERS = NUM_CORES * NUM_SUBCORES

BATCH = 16384
EMBED = 32
PACK = 128 // EMBED
B_PER_W = BATCH // NUM_WORKERS
CHUNK = 128
ID_ROWS = B_PER_W // CHUNK
HALF = B_PER_W // 2
HROWS = HALF // CHUNK


def _mf_body(uids_hbm, mids_hbm, ustage_hbm, mstage_hbm, out_hbm,
             uidv, midv, uix, mix, urows, mrows, outv, sem_u, sem_m):
  wid = lax.axis_index("s") * NUM_CORES + lax.axis_index("c")

  pltpu.sync_copy(uids_hbm.at[pl.ds(wid * ID_ROWS, ID_ROWS)], uidv)
  pltpu.sync_copy(mids_hbm.at[pl.ds(wid * ID_ROWS, ID_ROWS)], midv)

  def mkidx(r, _):
    for cc in range(CHUNK // LANES):
      col = cc * LANES
      uv = uidv[r, pl.ds(col, LANES)]
      mv = midv[r, pl.ds(col, LANES)]
      uix[r, pl.ds(col, LANES)] = ((uv >> 9) << 7) | (uv & 127)
      mix[r, pl.ds(col, LANES)] = ((mv >> 9) << 7) | (mv & 127)
    return 0

  lax.fori_loop(0, ID_ROWS, mkidx, 0)

  lane = lax.iota(jnp.int32, LANES)

  def run_pass(p, _):
    for rr in range(HROWS):
      pltpu.async_copy(
          ustage_hbm.at[uix.at[p * HROWS + rr]],
          urows.at[pl.ds(rr * CHUNK, CHUNK)], sem_u)
      pltpu.async_copy(
          mstage_hbm.at[mix.at[p * HROWS + rr]],
          mrows.at[pl.ds(rr * CHUNK, CHUNK)], sem_m)
    pltpu.make_async_copy(
        ustage_hbm.at[pl.ds(0, HALF)], urows, sem_u).wait()
    pltpu.make_async_copy(
        mstage_hbm.at[pl.ds(0, HALF)], mrows, sem_m).wait()

    def dot(g, _):
      b = p * HALF + g * LANES
      row = b // CHUNK
      col = b % CHUNK
      uvec = uidv[row, pl.ds(col, LANES)]
      mvec = midv[row, pl.ds(col, LANES)]
      acc = jnp.zeros((LANES,), jnp.float32)
      for k in range(LANES):
        e = g * LANES + k
        us = ((uvec[k] >> 7) & 3) * EMBED
        ms = ((mvec[k] >> 7) & 3) * EMBED
        u0 = urows[e, pl.ds(us, LANES)]
        u1 = urows[e, pl.ds(us + LANES, LANES)]
        m0 = mrows[e, pl.ds(ms, LANES)]
        m1 = mrows[e, pl.ds(ms + LANES, LANES)]
        s = u0 * m0 + u1 * m1
        tot = jnp.sum(s)
        acc = acc + jnp.where(lane == k, tot, jnp.float32(0))
      outv[pl.ds(b, LANES)] = acc
      return 0

    lax.fori_loop(0, HALF // LANES, dot, 0)
    return 0

  lax.fori_loop(0, 2, run_pass, 0)

  pltpu.sync_copy(outv, out_hbm.at[pl.ds(wid * B_PER_W, B_PER_W)])


RCW = 2048
QW = RCW // PACK


def _relayout_block(tab_ref, out_ref):
  x = tab_ref[...]
  for g in range(RCW // 512):
    base = g * 512
    y = jnp.concatenate(
        [x[:, base + a * 128:base + (a + 1) * 128] for a in range(PACK)],
        axis=0)
    out_ref[pl.ds(g * 128, 128), :] = jnp.transpose(y, (1, 0))


def _relayout(tab_t):
  n = tab_t.shape[1]
  grid = (n + RCW - 1) // RCW
  return pl.pallas_call(
      _relayout_block,
      grid=(grid,),
      in_specs=[pl.BlockSpec((EMBED, RCW), lambda i: (0, i))],
      out_specs=pl.BlockSpec((QW, 128), lambda i: (i, 0)),
      out_shape=jax.ShapeDtypeStruct((grid * QW, 128), jnp.float32),
  )(tab_t)


@jax.jit
def _mf(user_ids, movie_ids, user_table, movie_table):
  kern = pl.kernel(
      _mf_body,
      out_type=jax.ShapeDtypeStruct((BATCH,), jnp.float32),
      mesh=plsc.VectorSubcoreMesh(core_axis_name="c", subcore_axis_name="s"),
      scratch_types=[
          pltpu.VMEM((ID_ROWS, CHUNK), jnp.int32),
          pltpu.VMEM((ID_ROWS, CHUNK), jnp.int32),
          pltpu.VMEM((ID_ROWS, CHUNK), jnp.int32),
          pltpu.VMEM((ID_ROWS, CHUNK), jnp.int32),
          pltpu.VMEM((HALF, 128), jnp.float32),
          pltpu.VMEM((HALF, 128), jnp.float32),
          pltpu.VMEM((B_PER_W,), jnp.float32),
          pltpu.SemaphoreType.DMA,
          pltpu.SemaphoreType.DMA,
      ],
      compiler_params=pltpu.CompilerParams(needs_layout_passes=False),
  )
  uids = user_ids.astype(jnp.int32).reshape(BATCH // CHUNK, CHUNK)
  mids = movie_ids.astype(jnp.int32).reshape(BATCH // CHUNK, CHUNK)
  ustage = _relayout(jnp.swapaxes(user_table, 0, 1))
  mstage = _relayout(jnp.swapaxes(movie_table, 0, 1))
  return kern(uids, mids, ustage, mstage)


def kernel(user_ids, movie_ids, user_table, movie_table):
  return _mf(user_ids, movie_ids, user_table, movie_table)


# speedup vs baseline: 2.7162x; 2.1321x over previous
import functools

import jax
import jax.numpy as jnp
from jax import lax
from jax.experimental import pallas as pl
from jax.experimental.pallas import tpu as pltpu
from jax.experimental.pallas import tpu_sc as plsc

NUM_CORES = 2
NUM_SUBCORES = 16
LANES = 16
NUM_WORKERS = NUM_CORES * NUM_SUBCORES

BATCH = 16384
EMBED = 32
PACK = 128 // EMBED
B_PER_W = BATCH // NUM_WORKERS
CHUNK = 128
ID_ROWS = B_PER_W // CHUNK
HALF = B_PER_W // 2
HROWS = HALF // CHUNK


def _mf_body(uids_hbm, mids_hbm, ustage_hbm, mstage_hbm, out_hbm,
             uidv, midv, uix, mix, urows, mrows, outv, sem_u, sem_m):
  wid = lax.axis_index("s") * NUM_CORES + lax.axis_index("c")

  pltpu.sync_copy(uids_hbm.at[pl.ds(wid * ID_ROWS, ID_ROWS)], uidv)
  pltpu.sync_copy(mids_hbm.at[pl.ds(wid * ID_ROWS, ID_ROWS)], midv)

  def mkidx(r, _):
    for cc in range(CHUNK // LANES):
      col = cc * LANES
      uv = uidv[r, pl.ds(col, LANES)]
      mv = midv[r, pl.ds(col, LANES)]
      uix[r, pl.ds(col, LANES)] = ((uv >> 9) << 7) | (uv & 127)
      mix[r, pl.ds(col, LANES)] = ((mv >> 9) << 7) | (mv & 127)
    return 0

  lax.fori_loop(0, ID_ROWS, mkidx, 0)

  lane = lax.iota(jnp.int32, LANES)

  def run_pass(p, _):
    for rr in range(HROWS):
      pltpu.async_copy(
          ustage_hbm.at[uix.at[p * HROWS + rr]],
          urows.at[pl.ds(rr * CHUNK, CHUNK)], sem_u)
      pltpu.async_copy(
          mstage_hbm.at[mix.at[p * HROWS + rr]],
          mrows.at[pl.ds(rr * CHUNK, CHUNK)], sem_m)
    pltpu.make_async_copy(
        ustage_hbm.at[pl.ds(0, HALF)], urows, sem_u).wait()
    pltpu.make_async_copy(
        mstage_hbm.at[pl.ds(0, HALF)], mrows, sem_m).wait()

    def dot(g, _):
      b = p * HALF + g * LANES
      row = b // CHUNK
      col = b % CHUNK
      uvec = uidv[row, pl.ds(col, LANES)]
      mvec = midv[row, pl.ds(col, LANES)]
      acc = jnp.zeros((LANES,), jnp.float32)
      for k in range(LANES):
        e = g * LANES + k
        us = ((uvec[k] >> 7) & 3) * EMBED
        ms = ((mvec[k] >> 7) & 3) * EMBED
        u0 = urows[e, pl.ds(us, LANES)]
        u1 = urows[e, pl.ds(us + LANES, LANES)]
        m0 = mrows[e, pl.ds(ms, LANES)]
        m1 = mrows[e, pl.ds(ms + LANES, LANES)]
        s = u0 * m0 + u1 * m1
        tot = jnp.sum(s)
        acc = acc + jnp.where(lane == k, tot, jnp.float32(0))
      outv[pl.ds(b, LANES)] = acc
      return 0

    lax.fori_loop(0, HALF // LANES, dot, 0)
    return 0

  lax.fori_loop(0, 2, run_pass, 0)

  pltpu.sync_copy(outv, out_hbm.at[pl.ds(wid * B_PER_W, B_PER_W)])


RCW = 8192
QW = RCW // PACK


def _relayout_block(tab_ref, out_ref):
  x = tab_ref[...]
  for g in range(RCW // 512):
    base = g * 512
    y = jnp.concatenate(
        [x[:, base + a * 128:base + (a + 1) * 128] for a in range(PACK)],
        axis=0)
    out_ref[pl.ds(g * 128, 128), :] = jnp.transpose(y, (1, 0))


def _relayout(tab_t):
  n = tab_t.shape[1]
  grid = (n + RCW - 1) // RCW
  return pl.pallas_call(
      _relayout_block,
      grid=(grid,),
      in_specs=[pl.BlockSpec((EMBED, RCW), lambda i: (0, i))],
      out_specs=pl.BlockSpec((QW, 128), lambda i: (i, 0)),
      out_shape=jax.ShapeDtypeStruct((grid * QW, 128), jnp.float32),
  )(tab_t)


@jax.jit
def _mf(user_ids, movie_ids, user_table, movie_table):
  kern = pl.kernel(
      _mf_body,
      out_type=jax.ShapeDtypeStruct((BATCH,), jnp.float32),
      mesh=plsc.VectorSubcoreMesh(core_axis_name="c", subcore_axis_name="s"),
      scratch_types=[
          pltpu.VMEM((ID_ROWS, CHUNK), jnp.int32),
          pltpu.VMEM((ID_ROWS, CHUNK), jnp.int32),
          pltpu.VMEM((ID_ROWS, CHUNK), jnp.int32),
          pltpu.VMEM((ID_ROWS, CHUNK), jnp.int32),
          pltpu.VMEM((HALF, 128), jnp.float32),
          pltpu.VMEM((HALF, 128), jnp.float32),
          pltpu.VMEM((B_PER_W,), jnp.float32),
          pltpu.SemaphoreType.DMA,
          pltpu.SemaphoreType.DMA,
      ],
      compiler_params=pltpu.CompilerParams(needs_layout_passes=False),
  )
  uids = user_ids.astype(jnp.int32).reshape(BATCH // CHUNK, CHUNK)
  mids = movie_ids.astype(jnp.int32).reshape(BATCH // CHUNK, CHUNK)
  ustage = _relayout(jnp.swapaxes(user_table, 0, 1))
  mstage = _relayout(jnp.swapaxes(movie_table, 0, 1))
  return kern(uids, mids, ustage, mstage)


def kernel(user_ids, movie_ids, user_table, movie_table):
  return _mf(user_ids, movie_ids, user_table, movie_table)


# speedup vs baseline: 3.9009x; 1.4362x over previous
import functools

import jax
import jax.numpy as jnp
from jax import lax
from jax.experimental import pallas as pl
from jax.experimental.pallas import tpu as pltpu
from jax.experimental.pallas import tpu_sc as plsc

NUM_CORES = 2
NUM_SUBCORES = 16
LANES = 16
NUM_WORKERS = NUM_CORES * NUM_SUBCORES

BATCH = 16384
EMBED = 32
PACK = 128 // EMBED
B_PER_W = BATCH // NUM_WORKERS
CHUNK = 128
ID_ROWS = B_PER_W // CHUNK
HALF = B_PER_W // 2
HROWS = HALF // CHUNK


def _mf_body(uids_hbm, mids_hbm, ustage_hbm, mstage_hbm, out_hbm,
             uidv, midv, uix, mix, urows, mrows, outv, sem_u, sem_m):
  wid = lax.axis_index("s") * NUM_CORES + lax.axis_index("c")

  pltpu.sync_copy(uids_hbm.at[pl.ds(wid * ID_ROWS, ID_ROWS)], uidv)
  pltpu.sync_copy(mids_hbm.at[pl.ds(wid * ID_ROWS, ID_ROWS)], midv)

  def mkidx(r, _):
    for cc in range(CHUNK // LANES):
      col = cc * LANES
      uv = uidv[r, pl.ds(col, LANES)]
      mv = midv[r, pl.ds(col, LANES)]
      uix[r, pl.ds(col, LANES)] = ((uv >> 9) << 7) | (uv & 127)
      mix[r, pl.ds(col, LANES)] = ((mv >> 9) << 7) | (mv & 127)
    return 0

  lax.fori_loop(0, ID_ROWS, mkidx, 0)

  lane = lax.iota(jnp.int32, LANES)

  def run_pass(p, _):
    for rr in range(HROWS):
      pltpu.async_copy(
          ustage_hbm.at[uix.at[p * HROWS + rr]],
          urows.at[pl.ds(rr * CHUNK, CHUNK)], sem_u)
      pltpu.async_copy(
          mstage_hbm.at[mix.at[p * HROWS + rr]],
          mrows.at[pl.ds(rr * CHUNK, CHUNK)], sem_m)
    pltpu.make_async_copy(
        ustage_hbm.at[pl.ds(0, HALF)], urows, sem_u).wait()
    pltpu.make_async_copy(
        mstage_hbm.at[pl.ds(0, HALF)], mrows, sem_m).wait()

    def dot(g, _):
      b = p * HALF + g * LANES
      row = b // CHUNK
      col = b % CHUNK
      uvec = uidv[row, pl.ds(col, LANES)]
      mvec = midv[row, pl.ds(col, LANES)]
      acc = jnp.zeros((LANES,), jnp.float32)
      for k in range(LANES):
        e = g * LANES + k
        us = ((uvec[k] >> 7) & 3) * EMBED
        ms = ((mvec[k] >> 7) & 3) * EMBED
        u0 = urows[e, pl.ds(us, LANES)]
        u1 = urows[e, pl.ds(us + LANES, LANES)]
        m0 = mrows[e, pl.ds(ms, LANES)]
        m1 = mrows[e, pl.ds(ms + LANES, LANES)]
        s = u0 * m0 + u1 * m1
        tot = jnp.sum(s)
        acc = acc + jnp.where(lane == k, tot, jnp.float32(0))
      outv[pl.ds(b, LANES)] = acc
      return 0

    lax.fori_loop(0, HALF // LANES, dot, 0)
    return 0

  lax.fori_loop(0, 2, run_pass, 0)

  pltpu.sync_copy(outv, out_hbm.at[pl.ds(wid * B_PER_W, B_PER_W)])


RCW = 32768
QW = RCW // PACK


def _relayout_block(tab_ref, out_ref):
  x = tab_ref[...]
  for g in range(RCW // 512):
    base = g * 512
    y = jnp.concatenate(
        [x[:, base + a * 128:base + (a + 1) * 128] for a in range(PACK)],
        axis=0)
    out_ref[pl.ds(g * 128, 128), :] = jnp.transpose(y, (1, 0))


def _relayout(tab_t):
  n = tab_t.shape[1]
  grid = (n + RCW - 1) // RCW
  return pl.pallas_call(
      _relayout_block,
      grid=(grid,),
      in_specs=[pl.BlockSpec((EMBED, RCW), lambda i: (0, i))],
      out_specs=pl.BlockSpec((QW, 128), lambda i: (i, 0)),
      out_shape=jax.ShapeDtypeStruct((grid * QW, 128), jnp.float32),
  )(tab_t)


@jax.jit
def _mf(user_ids, movie_ids, user_table, movie_table):
  kern = pl.kernel(
      _mf_body,
      out_type=jax.ShapeDtypeStruct((BATCH,), jnp.float32),
      mesh=plsc.VectorSubcoreMesh(core_axis_name="c", subcore_axis_name="s"),
      scratch_types=[
          pltpu.VMEM((ID_ROWS, CHUNK), jnp.int32),
          pltpu.VMEM((ID_ROWS, CHUNK), jnp.int32),
          pltpu.VMEM((ID_ROWS, CHUNK), jnp.int32),
          pltpu.VMEM((ID_ROWS, CHUNK), jnp.int32),
          pltpu.VMEM((HALF, 128), jnp.float32),
          pltpu.VMEM((HALF, 128), jnp.float32),
          pltpu.VMEM((B_PER_W,), jnp.float32),
          pltpu.SemaphoreType.DMA,
          pltpu.SemaphoreType.DMA,
      ],
      compiler_params=pltpu.CompilerParams(needs_layout_passes=False),
  )
  uids = user_ids.astype(jnp.int32).reshape(BATCH // CHUNK, CHUNK)
  mids = movie_ids.astype(jnp.int32).reshape(BATCH // CHUNK, CHUNK)
  ustage = _relayout(jnp.swapaxes(user_table, 0, 1))
  mstage = _relayout(jnp.swapaxes(movie_table, 0, 1))
  return kern(uids, mids, ustage, mstage)


def kernel(user_ids, movie_ids, user_table, movie_table):
  return _mf(user_ids, movie_ids, user_table, movie_table)


# speedup vs baseline: 3.9665x; 1.0168x over previous
import functools

import jax
import jax.numpy as jnp
from jax import lax
from jax.experimental import pallas as pl
from jax.experimental.pallas import tpu as pltpu
from jax.experimental.pallas import tpu_sc as plsc

NUM_CORES = 2
NUM_SUBCORES = 16
LANES = 16
NUM_WORKERS = NUM_CORES * NUM_SUBCORES

BATCH = 16384
EMBED = 32
PACK = 128 // EMBED
B_PER_W = BATCH // NUM_WORKERS
CHUNK = 128
ID_ROWS = B_PER_W // CHUNK
HALF = B_PER_W // 2
HROWS = HALF // CHUNK


def _mf_body(uids_hbm, mids_hbm, ustage_hbm, mstage_hbm, out_hbm,
             uidv, midv, uix, mix, urows, mrows, outv, sem_u, sem_m):
  wid = lax.axis_index("s") * NUM_CORES + lax.axis_index("c")

  pltpu.sync_copy(uids_hbm.at[pl.ds(wid * ID_ROWS, ID_ROWS)], uidv)
  pltpu.sync_copy(mids_hbm.at[pl.ds(wid * ID_ROWS, ID_ROWS)], midv)

  def mkidx(r, _):
    for cc in range(CHUNK // LANES):
      col = cc * LANES
      uv = uidv[r, pl.ds(col, LANES)]
      mv = midv[r, pl.ds(col, LANES)]
      uix[r, pl.ds(col, LANES)] = ((uv >> 9) << 7) | (uv & 127)
      mix[r, pl.ds(col, LANES)] = ((mv >> 9) << 7) | (mv & 127)
    return 0

  lax.fori_loop(0, ID_ROWS, mkidx, 0)

  lane = lax.iota(jnp.int32, LANES)

  def run_pass(p, _):
    for rr in range(HROWS):
      pltpu.async_copy(
          ustage_hbm.at[uix.at[p * HROWS + rr]],
          urows.at[pl.ds(rr * CHUNK, CHUNK)], sem_u)
      pltpu.async_copy(
          mstage_hbm.at[mix.at[p * HROWS + rr]],
          mrows.at[pl.ds(rr * CHUNK, CHUNK)], sem_m)
    pltpu.make_async_copy(
        ustage_hbm.at[pl.ds(0, HALF)], urows, sem_u).wait()
    pltpu.make_async_copy(
        mstage_hbm.at[pl.ds(0, HALF)], mrows, sem_m).wait()

    def dot(g, _):
      b = p * HALF + g * LANES
      row = b // CHUNK
      col = b % CHUNK
      uvec = uidv[row, pl.ds(col, LANES)]
      mvec = midv[row, pl.ds(col, LANES)]
      acc = jnp.zeros((LANES,), jnp.float32)
      for k in range(LANES):
        e = g * LANES + k
        us = ((uvec[k] >> 7) & 3) * EMBED
        ms = ((mvec[k] >> 7) & 3) * EMBED
        u0 = urows[e, pl.ds(us, LANES)]
        u1 = urows[e, pl.ds(us + LANES, LANES)]
        m0 = mrows[e, pl.ds(ms, LANES)]
        m1 = mrows[e, pl.ds(ms + LANES, LANES)]
        s = u0 * m0 + u1 * m1
        tot = jnp.sum(s)
        acc = acc + jnp.where(lane == k, tot, jnp.float32(0))
      outv[pl.ds(b, LANES)] = acc
      return 0

    lax.fori_loop(0, HALF // LANES, dot, 0)
    return 0

  lax.fori_loop(0, 2, run_pass, 0)

  pltpu.sync_copy(outv, out_hbm.at[pl.ds(wid * B_PER_W, B_PER_W)])


RCW = 65536
QW = RCW // PACK


def _relayout_block(tab_ref, out_ref):
  x = tab_ref[...]
  for g in range(RCW // 512):
    base = g * 512
    y = jnp.concatenate(
        [x[:, base + a * 128:base + (a + 1) * 128] for a in range(PACK)],
        axis=0)
    out_ref[pl.ds(g * 128, 128), :] = jnp.transpose(y, (1, 0))


def _relayout(tab_t):
  n = tab_t.shape[1]
  grid = (n + RCW - 1) // RCW
  return pl.pallas_call(
      _relayout_block,
      grid=(grid,),
      in_specs=[pl.BlockSpec((EMBED, RCW), lambda i: (0, i))],
      out_specs=pl.BlockSpec((QW, 128), lambda i: (i, 0)),
      out_shape=jax.ShapeDtypeStruct((grid * QW, 128), jnp.float32),
  )(tab_t)


@jax.jit
def _mf(user_ids, movie_ids, user_table, movie_table):
  kern = pl.kernel(
      _mf_body,
      out_type=jax.ShapeDtypeStruct((BATCH,), jnp.float32),
      mesh=plsc.VectorSubcoreMesh(core_axis_name="c", subcore_axis_name="s"),
      scratch_types=[
          pltpu.VMEM((ID_ROWS, CHUNK), jnp.int32),
          pltpu.VMEM((ID_ROWS, CHUNK), jnp.int32),
          pltpu.VMEM((ID_ROWS, CHUNK), jnp.int32),
          pltpu.VMEM((ID_ROWS, CHUNK), jnp.int32),
          pltpu.VMEM((HALF, 128), jnp.float32),
          pltpu.VMEM((HALF, 128), jnp.float32),
          pltpu.VMEM((B_PER_W,), jnp.float32),
          pltpu.SemaphoreType.DMA,
          pltpu.SemaphoreType.DMA,
      ],
      compiler_params=pltpu.CompilerParams(needs_layout_passes=False),
  )
  uids = user_ids.astype(jnp.int32).reshape(BATCH // CHUNK, CHUNK)
  mids = movie_ids.astype(jnp.int32).reshape(BATCH // CHUNK, CHUNK)
  ustage = _relayout(jnp.swapaxes(user_table, 0, 1))
  mstage = _relayout(jnp.swapaxes(movie_table, 0, 1))
  return kern(uids, mids, ustage, mstage)


def kernel(user_ids, movie_ids, user_table, movie_table):
  return _mf(user_ids, movie_ids, user_table, movie_table)
